# baseline (speedup 1.0000x reference)
import functools

import jax
import jax.numpy as jnp
from jax.experimental import pallas as pl
from jax.experimental.pallas import tpu as pltpu

N = 10000
F = 256
NB = 625


def _mm_kernel(x_ref, w_ref, o_ref):
    o_ref[...] = jnp.dot(x_ref[...], w_ref[...],
                         preferred_element_type=jnp.float32).astype(jnp.bfloat16)


def _feat_mm(x, w):
    br = 1000
    return pl.pallas_call(
        _mm_kernel,
        grid=(N // br,),
        in_specs=[
            pl.BlockSpec((br, F), lambda i: (i, 0)),
            pl.BlockSpec((F, F), lambda i: (0, 0)),
        ],
        out_specs=pl.BlockSpec((br, F), lambda i: (i, 0)),
        out_shape=jax.ShapeDtypeStruct((N, F), jnp.bfloat16),
    )(x, w)


def _l1_kernel(adj_ref, y_ref, b_ref, bd_ref, h_ref, bs_ref):
    a16 = adj_ref[...].astype(jnp.bfloat16)
    acc = jnp.dot(a16, y_ref[...], preferred_element_type=jnp.float32)
    h_ref[...] = jnp.maximum(acc + b_ref[...], 0.0).astype(jnp.bfloat16)
    bs_ref[...] = jnp.dot(a16, bd_ref[...],
                          preferred_element_type=jnp.float32)


def _layer1(adj, y1, b1, bdiag):
    br = 200
    return pl.pallas_call(
        _l1_kernel,
        grid=(N // br,),
        in_specs=[
            pl.BlockSpec((br, N), lambda i: (i, 0)),
            pl.BlockSpec((N, F), lambda i: (0, 0)),
            pl.BlockSpec((1, F), lambda i: (0, 0)),
            pl.BlockSpec((N, NB), lambda i: (0, 0)),
        ],
        out_specs=[
            pl.BlockSpec((br, F), lambda i: (i, 0)),
            pl.BlockSpec((br, NB), lambda i: (i, 0)),
        ],
        out_shape=[
            jax.ShapeDtypeStruct((N, F), jnp.bfloat16),
            jax.ShapeDtypeStruct((N, NB), jnp.float32),
        ],
        compiler_params=pltpu.CompilerParams(
            dimension_semantics=("arbitrary",),
        ),
    )(adj, y1, b1, bdiag)


def _l2_kernel(adj_ref, y_ref, b_ref, o_ref):
    acc = jnp.dot(adj_ref[...].astype(jnp.bfloat16), y_ref[...],
                  preferred_element_type=jnp.float32)
    o_ref[...] = acc + b_ref[...]


def _layer2_dense(adj, y2, b2):
    br = 400
    return pl.pallas_call(
        _l2_kernel,
        grid=(N // br,),
        in_specs=[
            pl.BlockSpec((br, N), lambda i: (i, 0)),
            pl.BlockSpec((N, F), lambda i: (0, 0)),
            pl.BlockSpec((1, F), lambda i: (0, 0)),
        ],
        out_specs=pl.BlockSpec((br, F), lambda i: (i, 0)),
        out_shape=jax.ShapeDtypeStruct((N, F), jnp.float32),
        compiler_params=pltpu.CompilerParams(
            dimension_semantics=("arbitrary",),
        ),
    )(adj, y2, b2)


def kernel(x, adj, W1, b1, W2, b2):
    bdiag = (jnp.arange(N, dtype=jnp.int32)[:, None] // 16
             == jnp.arange(NB, dtype=jnp.int32)[None, :]).astype(jnp.bfloat16)
    y1 = _feat_mm(x, W1)
    hr, _bs = _layer1(adj, y1, b1.reshape(1, F), bdiag)
    y2 = _feat_mm(hr, W2)
    return _layer2_dense(adj, y2, b2.reshape(1, F))
